# baseline (speedup 1.0000x reference)
import jax
import jax.numpy as jnp
from jax.experimental import pallas as pl
from jax.experimental.pallas import tpu as pltpu

_R = 3
_M = 64
_N = 32768
_W = 8192
_CPR = _N // _W
_T = _R * _CPR
_NBUF = 12
_LA = 1


def _gather_body(mask_ref, x_ref, o_ref, *rest):
    bufs = rest[:_NBUF]
    in_sems = rest[_NBUF:2 * _NBUF]
    out_sems = rest[2 * _NBUF:3 * _NBUF]

    srcs = []
    for i in range(_R):
        count = 0
        src = 0
        for row in range(_R):
            keep = 1 - mask_ref[row]
            hit = jnp.logical_and(count == i, keep == 1)
            src = jnp.where(hit, row, src)
            count = count + keep
        srcs.append(src)

    loads = [None] * _T
    stores = [None] * _T
    for k in range(_T):
        r, j = divmod(k, _CPR)
        b = k % _NBUF
        if k >= _NBUF:
            stores[k - _NBUF].wait()
        loads[k] = pltpu.make_async_copy(
            x_ref.at[srcs[r], :, pl.ds(j * _W, _W)], bufs[b], in_sems[b]
        )
        loads[k].start()
        if k >= _LA:
            q = k - _LA
            rq, jq = divmod(q, _CPR)
            bq = q % _NBUF
            loads[q].wait()
            stores[q] = pltpu.make_async_copy(
                bufs[bq], o_ref.at[rq, :, pl.ds(jq * _W, _W)], out_sems[bq]
            )
            stores[q].start()
    for q in range(_T - _LA, _T):
        rq, jq = divmod(q, _CPR)
        bq = q % _NBUF
        loads[q].wait()
        stores[q] = pltpu.make_async_copy(
            bufs[bq], o_ref.at[rq, :, pl.ds(jq * _W, _W)], out_sems[bq]
        )
        stores[q].start()
    for q in range(_T - _NBUF, _T):
        stores[q].wait()


def kernel(x, bool_tensor):
    mask_i32 = bool_tensor.astype(jnp.int32)
    scratch = (
        [pltpu.VMEM((_M, _W), jnp.float32)] * _NBUF
        + [pltpu.SemaphoreType.DMA] * (2 * _NBUF)
    )
    out = pl.pallas_call(
        _gather_body,
        grid_spec=pltpu.PrefetchScalarGridSpec(
            num_scalar_prefetch=1,
            grid=(),
            in_specs=[pl.BlockSpec(memory_space=pl.ANY)],
            out_specs=pl.BlockSpec(memory_space=pl.ANY),
            scratch_shapes=scratch,
        ),
        out_shape=jax.ShapeDtypeStruct((_R, _M, _N), x.dtype),
    )(mask_i32, x)
    return out


# speedup vs baseline: 1.1635x; 1.1635x over previous
import jax
import jax.numpy as jnp
from jax.experimental import pallas as pl
from jax.experimental.pallas import tpu as pltpu

_R = 3
_M = 64
_N = 32768


def _copy_body(mask_ref, x_ref, o_ref):
    o_ref[...] = x_ref[...]


def _src_index_map(i, mask_ref):
    count = 0
    src = 0
    for row in range(_R):
        keep = 1 - mask_ref[row]
        hit = jnp.logical_and(count == i, keep == 1)
        src = jnp.where(hit, row, src)
        count = count + keep
    return (src, 0, 0)


def kernel(x, bool_tensor):
    mask_i32 = bool_tensor.astype(jnp.int32)
    out = pl.pallas_call(
        _copy_body,
        grid_spec=pltpu.PrefetchScalarGridSpec(
            num_scalar_prefetch=1,
            grid=(_R,),
            in_specs=[
                pl.BlockSpec((1, _M, _N), _src_index_map),
            ],
            out_specs=pl.BlockSpec((1, _M, _N), lambda i, m: (i, 0, 0)),
        ),
        out_shape=jax.ShapeDtypeStruct((_R, _M, _N), x.dtype),
    )(mask_i32, x)
    return out
